# baseline (speedup 1.0000x reference)
import functools

import jax
import jax.numpy as jnp
from jax import lax
from jax.experimental import pallas as pl
from jax.experimental.pallas import tpu as pltpu
from jax.experimental.pallas import tpu_sc as plsc

N1, N2, D = 8192, 1024, 128
E1, E2 = 262144, 32768
NC, NS = 2, 16
NW = NC * NS
TRASH = N2
ACC_ROWS = N2 + NS
E1W, E2W = E1 // NW, E2 // NW


def _mesh():
    return plsc.VectorSubcoreMesh(
        core_axis_name="c", subcore_axis_name="s",
        num_cores=NC, num_subcores=NS)


def _stage(e_per_w):
    return e_per_w + 256


def _offs_call(s1, d1, s2, d2):
    def tri(g):
        r = lax.broadcasted_iota(jnp.int32, (g, g), 0)
        c = lax.broadcasted_iota(jnp.int32, (g, g), 1)
        return jnp.where(r < c, 1.0, 0.0).astype(jnp.float32)

    def one(sv, dv, e_per_w, src_lim, dst_lim, wbias):
        g = e_per_w // 128
        m2 = (sv < src_lim) & (dv < dst_lim)
        mi = jnp.where(m2, 1.0, 0.0).astype(jnp.float32)
        excl = jnp.dot(mi, tri(128), preferred_element_type=jnp.float32)
        gs = jnp.sum(mi, axis=1)
        base = jnp.dot(gs, tri(g), preferred_element_type=jnp.float32)
        offs_val = (excl + base[:, None]).astype(jnp.int32)
        col = lax.broadcasted_iota(jnp.int32, (g, 128), 1)
        offs = jnp.where(m2, offs_val, (e_per_w + 128) + col) + wbias
        n = jnp.sum(gs).astype(jnp.int32)
        nch = (n + 127) >> 7
        pos = (lax.broadcasted_iota(jnp.int32, (8, 128), 0) * 128
               + lax.broadcasted_iota(jnp.int32, (8, 128), 1))
        cnt = jnp.where(pos == 0, n, jnp.where(pos == 1, nch, 0))
        return offs, cnt

    def body(s1_ref, d1_ref, s2_ref, d2_ref, o1_ref, c1_ref, o2_ref, c2_ref):
        w = pl.program_id(0)
        o1, c1 = one(s1_ref[0], d1_ref[0], E1W, N1, N2,
                     (w // NC) * _stage(E1W))
        o1_ref[0] = o1
        c1_ref[0] = c1
        o2, c2 = one(s2_ref[0], d2_ref[0], E2W, N2, N2,
                     (w // NC) * _stage(E2W))
        o2_ref[0] = o2
        c2_ref[0] = c2

    G1, G2 = E1W // 128, E2W // 128
    return pl.pallas_call(
        body,
        grid=(NW,),
        in_specs=[pl.BlockSpec((1, G1, 128), lambda w: (w, 0, 0)),
                  pl.BlockSpec((1, G1, 128), lambda w: (w, 0, 0)),
                  pl.BlockSpec((1, G2, 128), lambda w: (w, 0, 0)),
                  pl.BlockSpec((1, G2, 128), lambda w: (w, 0, 0))],
        out_specs=[pl.BlockSpec((1, G1, 128), lambda w: (w, 0, 0)),
                   pl.BlockSpec((1, 8, 128), lambda w: (w, 0, 0)),
                   pl.BlockSpec((1, G2, 128), lambda w: (w, 0, 0)),
                   pl.BlockSpec((1, 8, 128), lambda w: (w, 0, 0))],
        out_shape=[jax.ShapeDtypeStruct((NW, G1, 128), jnp.int32),
                   jax.ShapeDtypeStruct((NW, 8, 128), jnp.int32),
                   jax.ShapeDtypeStruct((NW, G2, 128), jnp.int32),
                   jax.ShapeDtypeStruct((NW, 8, 128), jnp.int32)],
    )(s1, d1, s2, d2)


def _deg_call(dst1_t, dst2_t, ones_t):
    c1, c2 = E1W // 128, E2W // 128
    r1, r2 = N1 // NS, N2 // NS

    @functools.partial(
        pl.kernel,
        out_type=[jax.ShapeDtypeStruct((NC, N1), jnp.float32),
                  jax.ShapeDtypeStruct((NC, N2), jnp.float32)],
        mesh=_mesh(),
        scratch_types=[pltpu.VMEM((c1, 128), jnp.int32),
                       pltpu.VMEM((c2, 128), jnp.int32),
                       pltpu.VMEM((c1, 128), jnp.float32),
                       pltpu.VMEM((r1,), jnp.float32),
                       pltpu.VMEM_SHARED((N1,), jnp.float32),
                       pltpu.VMEM_SHARED((N2,), jnp.float32),
                       pltpu.SemaphoreType.DMA])
    def k(dst1_h, dst2_h, ones_h, deg1p_h, deg2p_h,
          d1v, d2v, onesv, stg, deg1_s, deg2_s, sem):
        c = lax.axis_index("c")
        s = lax.axis_index("s")
        w = s * NC + c

        def zb(i, carry):
            stg[pl.ds(i * 16, 16)] = jnp.zeros((16,), jnp.float32)
            return carry
        lax.fori_loop(0, r1 // 16, zb, 0)
        pltpu.sync_copy(stg, deg1_s.at[pl.ds(s * r1, r1)])
        pltpu.sync_copy(stg.at[pl.ds(0, r2)], deg2_s.at[pl.ds(s * r2, r2)])
        pltpu.sync_copy(ones_h, onesv)
        pltpu.sync_copy(dst1_h.at[w], d1v)
        pltpu.sync_copy(dst2_h.at[w], d2v)
        plsc.subcore_barrier()

        def b1(j, carry):
            pltpu.sync_copy(onesv.at[j], deg1_s.at[d1v.at[j]], add=True)
            return carry
        lax.fori_loop(0, c1, b1, 0)

        def b2(j, carry):
            pltpu.sync_copy(onesv.at[j], deg2_s.at[d2v.at[j]], add=True)
            return carry
        lax.fori_loop(0, c2, b2, 0)
        plsc.subcore_barrier()
        pltpu.sync_copy(deg1_s.at[pl.ds(s * r1, r1)], stg)
        pltpu.sync_copy(stg, deg1p_h.at[c, pl.ds(s * r1, r1)])
        pltpu.sync_copy(deg2_s.at[pl.ds(s * r2, r2)], stg.at[pl.ds(0, r2)])
        pltpu.sync_copy(stg.at[pl.ds(0, r2)], deg2p_h.at[c, pl.ds(s * r2, r2)])

    return k(dst1_t, dst2_t, ones_t)


def _msgpass_call(src_t, dst_t, y, offs_t, cnt_t, e_per_w, v_rows):
    n_grp = e_per_w // 128
    stage = _stage(e_per_w)
    ra = ACC_ROWS // NS

    @functools.partial(
        pl.kernel,
        out_type=jax.ShapeDtypeStruct((NC, N2, D), jnp.float32),
        mesh=_mesh(),
        scratch_types=[pltpu.VMEM((e_per_w,), jnp.int32),
                       pltpu.VMEM((e_per_w,), jnp.int32),
                       pltpu.VMEM((n_grp, 128), jnp.int32),
                       pltpu.VMEM((128,), jnp.int32),
                       pltpu.VMEM((stage,), jnp.int32),
                       pltpu.VMEM((stage,), jnp.int32),
                       pltpu.VMEM((128, D), jnp.float32),
                       pltpu.VMEM((ra, D), jnp.float32),
                       pltpu.VMEM_SHARED((ACC_ROWS, D), jnp.float32),
                       pltpu.VMEM_SHARED((NS * stage,), jnp.int32),
                       pltpu.VMEM_SHARED((NS * stage,), jnp.int32),
                       pltpu.SemaphoreType.DMA])
    def k(src_h, dst_h, y_h, offs_h, cnt_h, accp_h,
          srcv, dstv, offsv, cntv, gsrc, gdst, rows_v, stg, acc_s,
          gsrc_s, gdst_s, sem):
        c = lax.axis_index("c")
        s = lax.axis_index("s")
        w = s * NC + c
        iot = lax.iota(jnp.int32, 16)

        def zb(i, carry):
            r = i // (D // 16)
            col = (i % (D // 16)) * 16
            stg[r, pl.ds(col, 16)] = jnp.zeros((16,), jnp.float32)
            return carry
        lax.fori_loop(0, ra * D // 16, zb, 0)
        pltpu.sync_copy(stg, acc_s.at[pl.ds(s * ra, ra)])
        pltpu.sync_copy(src_h.at[w], srcv)
        pltpu.sync_copy(dst_h.at[w], dstv)
        pltpu.sync_copy(offs_h.at[w], offsv)
        pltpu.sync_copy(cnt_h.at[w], cntv)
        plsc.subcore_barrier()
        cvec = cntv[pl.ds(0, 16)]
        n = cvec[0]
        nch = cvec[1]

        def gbody(g, carry):
            pltpu.sync_copy(srcv.at[pl.ds(g * 128, 128)],
                            gsrc_s.at[offsv.at[g]])
            pltpu.sync_copy(dstv.at[pl.ds(g * 128, 128)],
                            gdst_s.at[offsv.at[g]])
            return carry
        lax.fori_loop(0, n_grp, gbody, 0)

        pltpu.sync_copy(gsrc_s.at[pl.ds(s * stage, stage)], gsrc)
        pltpu.sync_copy(gdst_s.at[pl.ds(s * stage, stage)], gdst)

        for kk in range(8):
            gsrc[pl.ds(n + kk * 16, 16)] = iot * (v_rows // 16) + kk * 7
            gdst[pl.ds(n + kk * 16, 16)] = jnp.full((16,), TRASH, jnp.int32) + s
        plsc.subcore_barrier()

        def cbody(j, carry):
            pltpu.async_copy(
                y_h.at[gsrc.at[pl.ds(j * 128, 128)]], rows_v, sem).wait()
            for kk in range(8):
                didx = gdst[pl.ds(j * 128 + kk * 16, 16)]
                pltpu.sync_copy(rows_v.at[pl.ds(kk * 16, 16)],
                                acc_s.at[didx], add=True)
            return carry
        lax.fori_loop(0, nch, cbody, 0)
        plsc.subcore_barrier()
        ro = N2 // NS
        pltpu.sync_copy(acc_s.at[pl.ds(s * ro, ro)], stg.at[pl.ds(0, ro)])
        pltpu.sync_copy(stg.at[pl.ds(0, ro)], accp_h.at[c, pl.ds(s * ro, ro)])

    return k(src_t, dst_t, y, offs_t, cnt_t)


def _tc_a(x8k, W1, deg1p):
    def body(x_ref, w_ref, d_ref, y_ref):
        dis = lax.rsqrt(d_ref[0, :] + d_ref[1, :] + 2.0)
        y_ref[...] = dis[:, None] * jnp.dot(
            x_ref[...], w_ref[...], preferred_element_type=jnp.float32)
    blk = 512
    return pl.pallas_call(
        body,
        grid=(N1 // blk,),
        in_specs=[pl.BlockSpec((blk, D), lambda i: (i, 0)),
                  pl.BlockSpec((D, D), lambda i: (0, 0)),
                  pl.BlockSpec((2, blk), lambda i: (0, i))],
        out_specs=pl.BlockSpec((blk, D), lambda i: (i, 0)),
        out_shape=jax.ShapeDtypeStruct((N1, D), jnp.float32))(x8k, W1, deg1p)


def _tc_b(a1p, y1k, d1k, b1, W2, d2p):
    def body(a_ref, y_ref, d1_ref, b_ref, w_ref, d2_ref, o_ref):
        dis1 = lax.rsqrt(d1_ref[0, :] + d1_ref[1, :] + 2.0)
        acc = a_ref[0] + a_ref[1] + 2.0 * y_ref[...]
        h = jnp.maximum(dis1[:, None] * acc + b_ref[...], 0.0)
        dis2 = lax.rsqrt(d2_ref[0, :] + d2_ref[1, :] + 1.0)
        o_ref[...] = dis2[:, None] * jnp.dot(
            h, w_ref[...], preferred_element_type=jnp.float32)
    return pl.pallas_call(
        body,
        out_shape=jax.ShapeDtypeStruct((N2, D), jnp.float32),
    )(a1p, y1k, d1k, b1, W2, d2p)


def _tc_c(a2p, y2, d2p, b2):
    def body(a_ref, y_ref, d_ref, b_ref, o_ref):
        dis2 = lax.rsqrt(d_ref[0, :] + d_ref[1, :] + 1.0)
        o_ref[...] = dis2[:, None] * (a_ref[0] + a_ref[1] + y_ref[...]) \
            + b_ref[...]
    return pl.pallas_call(
        body,
        out_shape=jax.ShapeDtypeStruct((N2, D), jnp.float32),
    )(a2p, y2, d2p, b2)


def kernel(x, edge_index1, edge_index2, W1, b1, W2, b2):
    ei1 = edge_index1.astype(jnp.int32)
    ei2 = edge_index2.astype(jnp.int32)
    src1, dst1 = ei1[0], ei1[1]
    src2, dst2 = ei2[0], ei2[1]
    x8k = x[:N1]

    ones_t = jnp.ones((E1W // 128, 128), jnp.float32)

    offs1, cnt1, offs2, cnt2 = _offs_call(
        src1.reshape(NW, E1W // 128, 128), dst1.reshape(NW, E1W // 128, 128),
        src2.reshape(NW, E2W // 128, 128), dst2.reshape(NW, E2W // 128, 128))
    deg1p, deg2p = _deg_call(
        dst1.reshape(NW, E1W // 128, 128),
        dst2.reshape(NW, E2W // 128, 128), ones_t)
    y1 = _tc_a(x8k, W1, deg1p)
    a1p = _msgpass_call(src1.reshape(NW, E1W), dst1.reshape(NW, E1W),
                        y1, offs1, cnt1.reshape(NW, 1024)[:, :128], E1W, N1)
    y2 = _tc_b(a1p, y1[:N2], deg1p[:, :N2], b1.reshape(1, D), W2, deg2p)
    a2p = _msgpass_call(src2.reshape(NW, E2W), dst2.reshape(NW, E2W),
                        y2, offs2, cnt2.reshape(NW, 1024)[:, :128], E2W, N2)
    return _tc_c(a2p, y2, deg2p, b2.reshape(1, D))


# speedup vs baseline: 1.0378x; 1.0378x over previous
import functools

import jax
import jax.numpy as jnp
from jax import lax
from jax.experimental import pallas as pl
from jax.experimental.pallas import tpu as pltpu
from jax.experimental.pallas import tpu_sc as plsc

N1, N2, D = 8192, 1024, 128
E1, E2 = 262144, 32768
NC, NS = 2, 16
NW = NC * NS
TRASH = N2
ACC_ROWS = N2 + NS
E1W, E2W = E1 // NW, E2 // NW


def _mesh():
    return plsc.VectorSubcoreMesh(
        core_axis_name="c", subcore_axis_name="s",
        num_cores=NC, num_subcores=NS)


def _stage(e_per_w):
    return e_per_w + 256


def _offs_call(s1, d1, s2, d2):
    def tri(g):
        r = lax.broadcasted_iota(jnp.int32, (g, g), 0)
        c = lax.broadcasted_iota(jnp.int32, (g, g), 1)
        return jnp.where(r < c, 1.0, 0.0).astype(jnp.float32)

    def one(sv, dv, e_per_w, src_lim, dst_lim, wbias):
        g = e_per_w // 128
        m2 = (sv < src_lim) & (dv < dst_lim)
        mi = jnp.where(m2, 1.0, 0.0).astype(jnp.float32)
        excl = jnp.dot(mi, tri(128), preferred_element_type=jnp.float32)
        gs = jnp.sum(mi, axis=1)
        base = jnp.dot(gs, tri(g), preferred_element_type=jnp.float32)
        offs_val = (excl + base[:, None]).astype(jnp.int32)
        col = lax.broadcasted_iota(jnp.int32, (g, 128), 1)
        offs = jnp.where(m2, offs_val, (e_per_w + 128) + col) + wbias
        n = jnp.sum(gs).astype(jnp.int32)
        nch = (n + 127) >> 7
        pos = (lax.broadcasted_iota(jnp.int32, (8, 128), 0) * 128
               + lax.broadcasted_iota(jnp.int32, (8, 128), 1))
        cnt = jnp.where(pos == 0, n, jnp.where(pos == 1, nch, 0))
        return offs, cnt

    def body(s1_ref, d1_ref, s2_ref, d2_ref, o1_ref, c1_ref, o2_ref, c2_ref):
        w = pl.program_id(0)
        o1, c1 = one(s1_ref[0], d1_ref[0], E1W, N1, N2,
                     (w // NC) * _stage(E1W))
        o1_ref[0] = o1
        c1_ref[0] = c1
        o2, c2 = one(s2_ref[0], d2_ref[0], E2W, N2, N2,
                     (w // NC) * _stage(E2W))
        o2_ref[0] = o2
        c2_ref[0] = c2

    G1, G2 = E1W // 128, E2W // 128
    return pl.pallas_call(
        body,
        grid=(NW,),
        in_specs=[pl.BlockSpec((1, G1, 128), lambda w: (w, 0, 0)),
                  pl.BlockSpec((1, G1, 128), lambda w: (w, 0, 0)),
                  pl.BlockSpec((1, G2, 128), lambda w: (w, 0, 0)),
                  pl.BlockSpec((1, G2, 128), lambda w: (w, 0, 0))],
        out_specs=[pl.BlockSpec((1, G1, 128), lambda w: (w, 0, 0)),
                   pl.BlockSpec((1, 8, 128), lambda w: (w, 0, 0)),
                   pl.BlockSpec((1, G2, 128), lambda w: (w, 0, 0)),
                   pl.BlockSpec((1, 8, 128), lambda w: (w, 0, 0))],
        out_shape=[jax.ShapeDtypeStruct((NW, G1, 128), jnp.int32),
                   jax.ShapeDtypeStruct((NW, 8, 128), jnp.int32),
                   jax.ShapeDtypeStruct((NW, G2, 128), jnp.int32),
                   jax.ShapeDtypeStruct((NW, 8, 128), jnp.int32)],
    )(s1, d1, s2, d2)


def _deg_call(dst1_t, dst2_t, ones_t):
    c1, c2 = E1W // 128, E2W // 128
    r1, r2 = N1 // NS, N2 // NS

    @functools.partial(
        pl.kernel,
        out_type=[jax.ShapeDtypeStruct((NC, N1), jnp.float32),
                  jax.ShapeDtypeStruct((NC, N2), jnp.float32)],
        mesh=_mesh(),
        scratch_types=[pltpu.VMEM((c1, 128), jnp.int32),
                       pltpu.VMEM((c2, 128), jnp.int32),
                       pltpu.VMEM((c1, 128), jnp.float32),
                       pltpu.VMEM((r1,), jnp.float32),
                       pltpu.VMEM_SHARED((N1,), jnp.float32),
                       pltpu.VMEM_SHARED((N2,), jnp.float32),
                       pltpu.SemaphoreType.DMA])
    def k(dst1_h, dst2_h, ones_h, deg1p_h, deg2p_h,
          d1v, d2v, onesv, stg, deg1_s, deg2_s, sem):
        c = lax.axis_index("c")
        s = lax.axis_index("s")
        w = s * NC + c

        def zb(i, carry):
            stg[pl.ds(i * 16, 16)] = jnp.zeros((16,), jnp.float32)
            return carry
        lax.fori_loop(0, r1 // 16, zb, 0)
        pltpu.sync_copy(stg, deg1_s.at[pl.ds(s * r1, r1)])
        pltpu.sync_copy(stg.at[pl.ds(0, r2)], deg2_s.at[pl.ds(s * r2, r2)])
        pltpu.sync_copy(ones_h, onesv)
        pltpu.sync_copy(dst1_h.at[w], d1v)
        pltpu.sync_copy(dst2_h.at[w], d2v)
        plsc.subcore_barrier()

        def b1(j, carry):
            pltpu.sync_copy(onesv.at[j], deg1_s.at[d1v.at[j]], add=True)
            return carry
        lax.fori_loop(0, c1, b1, 0)

        def b2(j, carry):
            pltpu.sync_copy(onesv.at[j], deg2_s.at[d2v.at[j]], add=True)
            return carry
        lax.fori_loop(0, c2, b2, 0)
        plsc.subcore_barrier()
        pltpu.sync_copy(deg1_s.at[pl.ds(s * r1, r1)], stg)
        pltpu.sync_copy(stg, deg1p_h.at[c, pl.ds(s * r1, r1)])
        pltpu.sync_copy(deg2_s.at[pl.ds(s * r2, r2)], stg.at[pl.ds(0, r2)])
        pltpu.sync_copy(stg.at[pl.ds(0, r2)], deg2p_h.at[c, pl.ds(s * r2, r2)])

    return k(dst1_t, dst2_t, ones_t)


def _msgpass_call(src_t, dst_t, y, offs_t, cnt_t, e_per_w, v_rows):
    n_grp = e_per_w // 128
    stage = _stage(e_per_w)
    ra = ACC_ROWS // NS

    @functools.partial(
        pl.kernel,
        out_type=jax.ShapeDtypeStruct((NC, N2, D), jnp.float32),
        mesh=_mesh(),
        scratch_types=[pltpu.VMEM((e_per_w,), jnp.int32),
                       pltpu.VMEM((e_per_w,), jnp.int32),
                       pltpu.VMEM((n_grp, 128), jnp.int32),
                       pltpu.VMEM((128,), jnp.int32),
                       pltpu.VMEM((stage,), jnp.int32),
                       pltpu.VMEM((stage,), jnp.int32),
                       pltpu.VMEM((128, D), jnp.float32),
                       pltpu.VMEM((ra, D), jnp.float32),
                       pltpu.VMEM_SHARED((ACC_ROWS, D), jnp.float32),
                       pltpu.VMEM_SHARED((NS * stage,), jnp.int32),
                       pltpu.VMEM_SHARED((NS * stage,), jnp.int32),
                       pltpu.SemaphoreType.DMA,
                       pltpu.SemaphoreType.DMA])
    def k(src_h, dst_h, y_h, offs_h, cnt_h, accp_h,
          srcv, dstv, offsv, cntv, gsrc, gdst, rows_v, stg, acc_s,
          gsrc_s, gdst_s, sem, sem2):
        c = lax.axis_index("c")
        s = lax.axis_index("s")
        w = s * NC + c
        iot = lax.iota(jnp.int32, 16)

        def zb(i, carry):
            r = i // (D // 16)
            col = (i % (D // 16)) * 16
            stg[r, pl.ds(col, 16)] = jnp.zeros((16,), jnp.float32)
            return carry
        lax.fori_loop(0, ra * D // 16, zb, 0)
        pltpu.sync_copy(stg, acc_s.at[pl.ds(s * ra, ra)])
        pltpu.sync_copy(src_h.at[w], srcv)
        pltpu.sync_copy(dst_h.at[w], dstv)
        pltpu.sync_copy(offs_h.at[w], offsv)
        pltpu.sync_copy(cnt_h.at[w], cntv)
        plsc.subcore_barrier()
        cvec = cntv[pl.ds(0, 16)]
        n = cvec[0]
        nch = cvec[1]

        def gbody(g, carry):
            d1 = pltpu.async_copy(
                srcv.at[pl.ds(g * 128, 128)], gsrc_s.at[offsv.at[g]], sem)
            d2 = pltpu.async_copy(
                dstv.at[pl.ds(g * 128, 128)], gdst_s.at[offsv.at[g]], sem2)
            d1.wait()
            d2.wait()
            return carry
        lax.fori_loop(0, n_grp, gbody, 0)

        pltpu.sync_copy(gsrc_s.at[pl.ds(s * stage, stage)], gsrc)
        pltpu.sync_copy(gdst_s.at[pl.ds(s * stage, stage)], gdst)

        for kk in range(8):
            gsrc[pl.ds(n + kk * 16, 16)] = iot * (v_rows // 16) + kk * 7
            gdst[pl.ds(n + kk * 16, 16)] = jnp.full((16,), TRASH, jnp.int32) + s
        plsc.subcore_barrier()

        def cbody(j, carry):
            pltpu.async_copy(
                y_h.at[gsrc.at[pl.ds(j * 128, 128)]], rows_v, sem).wait()
            for kk in range(8):
                didx = gdst[pl.ds(j * 128 + kk * 16, 16)]
                pltpu.sync_copy(rows_v.at[pl.ds(kk * 16, 16)],
                                acc_s.at[didx], add=True)
            return carry
        lax.fori_loop(0, nch, cbody, 0)
        plsc.subcore_barrier()
        ro = N2 // NS
        pltpu.sync_copy(acc_s.at[pl.ds(s * ro, ro)], stg.at[pl.ds(0, ro)])
        pltpu.sync_copy(stg.at[pl.ds(0, ro)], accp_h.at[c, pl.ds(s * ro, ro)])

    return k(src_t, dst_t, y, offs_t, cnt_t)


def _tc_a(x8k, W1, deg1p):
    def body(x_ref, w_ref, d_ref, y_ref):
        dis = lax.rsqrt(d_ref[0, :] + d_ref[1, :] + 2.0)
        y_ref[...] = dis[:, None] * jnp.dot(
            x_ref[...], w_ref[...], preferred_element_type=jnp.float32)
    blk = 512
    return pl.pallas_call(
        body,
        grid=(N1 // blk,),
        in_specs=[pl.BlockSpec((blk, D), lambda i: (i, 0)),
                  pl.BlockSpec((D, D), lambda i: (0, 0)),
                  pl.BlockSpec((2, blk), lambda i: (0, i))],
        out_specs=pl.BlockSpec((blk, D), lambda i: (i, 0)),
        out_shape=jax.ShapeDtypeStruct((N1, D), jnp.float32))(x8k, W1, deg1p)


def _tc_b(a1p, y1k, d1k, b1, W2, d2p):
    def body(a_ref, y_ref, d1_ref, b_ref, w_ref, d2_ref, o_ref):
        dis1 = lax.rsqrt(d1_ref[0, :] + d1_ref[1, :] + 2.0)
        acc = a_ref[0] + a_ref[1] + 2.0 * y_ref[...]
        h = jnp.maximum(dis1[:, None] * acc + b_ref[...], 0.0)
        dis2 = lax.rsqrt(d2_ref[0, :] + d2_ref[1, :] + 1.0)
        o_ref[...] = dis2[:, None] * jnp.dot(
            h, w_ref[...], preferred_element_type=jnp.float32)
    return pl.pallas_call(
        body,
        out_shape=jax.ShapeDtypeStruct((N2, D), jnp.float32),
    )(a1p, y1k, d1k, b1, W2, d2p)


def _tc_c(a2p, y2, d2p, b2):
    def body(a_ref, y_ref, d_ref, b_ref, o_ref):
        dis2 = lax.rsqrt(d_ref[0, :] + d_ref[1, :] + 1.0)
        o_ref[...] = dis2[:, None] * (a_ref[0] + a_ref[1] + y_ref[...]) \
            + b_ref[...]
    return pl.pallas_call(
        body,
        out_shape=jax.ShapeDtypeStruct((N2, D), jnp.float32),
    )(a2p, y2, d2p, b2)


def kernel(x, edge_index1, edge_index2, W1, b1, W2, b2):
    ei1 = edge_index1.astype(jnp.int32)
    ei2 = edge_index2.astype(jnp.int32)
    src1, dst1 = ei1[0], ei1[1]
    src2, dst2 = ei2[0], ei2[1]
    x8k = x[:N1]

    ones_t = jnp.ones((E1W // 128, 128), jnp.float32)

    offs1, cnt1, offs2, cnt2 = _offs_call(
        src1.reshape(NW, E1W // 128, 128), dst1.reshape(NW, E1W // 128, 128),
        src2.reshape(NW, E2W // 128, 128), dst2.reshape(NW, E2W // 128, 128))
    deg1p, deg2p = _deg_call(
        dst1.reshape(NW, E1W // 128, 128),
        dst2.reshape(NW, E2W // 128, 128), ones_t)
    y1 = _tc_a(x8k, W1, deg1p)
    a1p = _msgpass_call(src1.reshape(NW, E1W), dst1.reshape(NW, E1W),
                        y1, offs1, cnt1.reshape(NW, 1024)[:, :128], E1W, N1)
    y2 = _tc_b(a1p, y1[:N2], deg1p[:, :N2], b1.reshape(1, D), W2, deg2p)
    a2p = _msgpass_call(src2.reshape(NW, E2W), dst2.reshape(NW, E2W),
                        y2, offs2, cnt2.reshape(NW, 1024)[:, :128], E2W, N2)
    return _tc_c(a2p, y2, deg2p, b2.reshape(1, D))


# speedup vs baseline: 1.0483x; 1.0101x over previous
import functools

import jax
import jax.numpy as jnp
from jax import lax
from jax.experimental import pallas as pl
from jax.experimental.pallas import tpu as pltpu
from jax.experimental.pallas import tpu_sc as plsc

N1, N2, D = 8192, 1024, 128
E1, E2 = 262144, 32768
NC, NS = 2, 16
NW = NC * NS
TRASH = N2
ACC_ROWS = N2 + NS
E1W, E2W = E1 // NW, E2 // NW


def _mesh():
    return plsc.VectorSubcoreMesh(
        core_axis_name="c", subcore_axis_name="s",
        num_cores=NC, num_subcores=NS)


def _stage(e_per_w):
    return e_per_w + 256


def _tca_offs_call(x8k, W1, deg1p, s1, d1, s2, d2):
    def tri(g):
        r = lax.broadcasted_iota(jnp.int32, (g, g), 0)
        c = lax.broadcasted_iota(jnp.int32, (g, g), 1)
        return jnp.where(r < c, 1.0, 0.0).astype(jnp.float32)

    def one(sv, dv, e_per_w, src_lim, dst_lim, wbias):
        g = e_per_w // 128
        m2 = (sv < src_lim) & (dv < dst_lim)
        mi = jnp.where(m2, 1.0, 0.0).astype(jnp.float32)
        excl = jnp.dot(mi, tri(128), preferred_element_type=jnp.float32)
        gs = jnp.sum(mi, axis=1)
        base = jnp.dot(gs, tri(g), preferred_element_type=jnp.float32)
        offs_val = (excl + base[:, None]).astype(jnp.int32)
        col = lax.broadcasted_iota(jnp.int32, (g, 128), 1)
        offs = jnp.where(m2, offs_val, (e_per_w + 128) + col) + wbias
        n = jnp.sum(gs).astype(jnp.int32)
        nch = (n + 127) >> 7
        pos = (lax.broadcasted_iota(jnp.int32, (8, 128), 0) * 128
               + lax.broadcasted_iota(jnp.int32, (8, 128), 1))
        cnt = jnp.where(pos == 0, n, jnp.where(pos == 1, nch, 0))
        return offs, cnt

    def body(x_ref, w_ref, dg_ref, s1_ref, d1_ref, s2_ref, d2_ref,
             y_ref, o1_ref, c1_ref, o2_ref, c2_ref):
        w = pl.program_id(0)
        dis = lax.rsqrt(dg_ref[0, :] + dg_ref[1, :] + 2.0)
        y_ref[...] = dis[:, None] * jnp.dot(
            x_ref[...], w_ref[...], preferred_element_type=jnp.float32)
        o1, c1 = one(s1_ref[0], d1_ref[0], E1W, N1, N2,
                     (w // NC) * _stage(E1W))
        o1_ref[0] = o1
        c1_ref[0] = c1
        o2, c2 = one(s2_ref[0], d2_ref[0], E2W, N2, N2,
                     (w // NC) * _stage(E2W))
        o2_ref[0] = o2
        c2_ref[0] = c2

    G1, G2 = E1W // 128, E2W // 128
    blk = N1 // NW
    return pl.pallas_call(
        body,
        grid=(NW,),
        in_specs=[pl.BlockSpec((blk, D), lambda w: (w, 0)),
                  pl.BlockSpec((D, D), lambda w: (0, 0)),
                  pl.BlockSpec((2, blk), lambda w: (0, w)),
                  pl.BlockSpec((1, G1, 128), lambda w: (w, 0, 0)),
                  pl.BlockSpec((1, G1, 128), lambda w: (w, 0, 0)),
                  pl.BlockSpec((1, G2, 128), lambda w: (w, 0, 0)),
                  pl.BlockSpec((1, G2, 128), lambda w: (w, 0, 0))],
        out_specs=[pl.BlockSpec((blk, D), lambda w: (w, 0)),
                   pl.BlockSpec((1, G1, 128), lambda w: (w, 0, 0)),
                   pl.BlockSpec((1, 8, 128), lambda w: (w, 0, 0)),
                   pl.BlockSpec((1, G2, 128), lambda w: (w, 0, 0)),
                   pl.BlockSpec((1, 8, 128), lambda w: (w, 0, 0))],
        out_shape=[jax.ShapeDtypeStruct((N1, D), jnp.float32),
                   jax.ShapeDtypeStruct((NW, G1, 128), jnp.int32),
                   jax.ShapeDtypeStruct((NW, 8, 128), jnp.int32),
                   jax.ShapeDtypeStruct((NW, G2, 128), jnp.int32),
                   jax.ShapeDtypeStruct((NW, 8, 128), jnp.int32)],
    )(x8k, W1, deg1p, s1, d1, s2, d2)


def _deg_call(dst1_t, dst2_t, ones_t):
    c1, c2 = E1W // 128, E2W // 128
    r1, r2 = N1 // NS, N2 // NS

    @functools.partial(
        pl.kernel,
        out_type=[jax.ShapeDtypeStruct((NC, N1), jnp.float32),
                  jax.ShapeDtypeStruct((NC, N2), jnp.float32)],
        mesh=_mesh(),
        scratch_types=[pltpu.VMEM((c1, 128), jnp.int32),
                       pltpu.VMEM((c2, 128), jnp.int32),
                       pltpu.VMEM((c1, 128), jnp.float32),
                       pltpu.VMEM((r1,), jnp.float32),
                       pltpu.VMEM_SHARED((N1,), jnp.float32),
                       pltpu.VMEM_SHARED((N2,), jnp.float32),
                       pltpu.SemaphoreType.DMA,
                       pltpu.SemaphoreType.DMA])
    def k(dst1_h, dst2_h, ones_h, deg1p_h, deg2p_h,
          d1v, d2v, onesv, stg, deg1_s, deg2_s, sem, sem2):
        c = lax.axis_index("c")
        s = lax.axis_index("s")
        w = s * NC + c

        def zb(i, carry):
            stg[pl.ds(i * 16, 16)] = jnp.zeros((16,), jnp.float32)
            return carry
        lax.fori_loop(0, r1 // 16, zb, 0)
        pltpu.sync_copy(stg, deg1_s.at[pl.ds(s * r1, r1)])
        pltpu.sync_copy(stg.at[pl.ds(0, r2)], deg2_s.at[pl.ds(s * r2, r2)])
        pltpu.sync_copy(ones_h, onesv)
        pltpu.sync_copy(dst1_h.at[w], d1v)
        pltpu.sync_copy(dst2_h.at[w], d2v)
        plsc.subcore_barrier()

        def b1(j, carry):
            pltpu.sync_copy(onesv.at[j], deg1_s.at[d1v.at[j]], add=True)
            return carry
        lax.fori_loop(0, c1, b1, 0)

        def b2(j, carry):
            pltpu.sync_copy(onesv.at[j], deg2_s.at[d2v.at[j]], add=True)
            return carry
        lax.fori_loop(0, c2, b2, 0)
        plsc.subcore_barrier()
        pltpu.sync_copy(deg1_s.at[pl.ds(s * r1, r1)], stg)
        pltpu.sync_copy(stg, deg1p_h.at[c, pl.ds(s * r1, r1)])
        pltpu.sync_copy(deg2_s.at[pl.ds(s * r2, r2)], stg.at[pl.ds(0, r2)])
        pltpu.sync_copy(stg.at[pl.ds(0, r2)], deg2p_h.at[c, pl.ds(s * r2, r2)])

    return k(dst1_t, dst2_t, ones_t)


def _msgpass_call(src_t, dst_t, y, offs_t, cnt_t, e_per_w, v_rows):
    n_grp = e_per_w // 128
    stage = _stage(e_per_w)
    ra = ACC_ROWS // NS

    @functools.partial(
        pl.kernel,
        out_type=jax.ShapeDtypeStruct((NC, N2, D), jnp.float32),
        mesh=_mesh(),
        scratch_types=[pltpu.VMEM((e_per_w,), jnp.int32),
                       pltpu.VMEM((e_per_w,), jnp.int32),
                       pltpu.VMEM((n_grp, 128), jnp.int32),
                       pltpu.VMEM((128,), jnp.int32),
                       pltpu.VMEM((stage,), jnp.int32),
                       pltpu.VMEM((stage,), jnp.int32),
                       pltpu.VMEM((128, D), jnp.float32),
                       pltpu.VMEM((ra, D), jnp.float32),
                       pltpu.VMEM_SHARED((ACC_ROWS, D), jnp.float32),
                       pltpu.VMEM_SHARED((NS * stage,), jnp.int32),
                       pltpu.VMEM_SHARED((NS * stage,), jnp.int32),
                       pltpu.SemaphoreType.DMA,
                       pltpu.SemaphoreType.DMA])
    def k(src_h, dst_h, y_h, offs_h, cnt_h, zacc_h, accp_h,
          srcv, dstv, offsv, cntv, gsrc, gdst, rows_v, stg, acc_s,
          gsrc_s, gdst_s, sem, sem2):
        c = lax.axis_index("c")
        s = lax.axis_index("s")
        w = s * NC + c
        iot = lax.iota(jnp.int32, 16)

        pltpu.sync_copy(zacc_h, stg)
        pltpu.sync_copy(stg, acc_s.at[pl.ds(s * ra, ra)])
        p1 = pltpu.async_copy(src_h.at[w], srcv, sem)
        p2 = pltpu.async_copy(dst_h.at[w], dstv, sem2)
        p1.wait()
        p2.wait()
        p1 = pltpu.async_copy(offs_h.at[w], offsv, sem)
        p2 = pltpu.async_copy(cnt_h.at[w], cntv, sem2)
        p1.wait()
        p2.wait()
        plsc.subcore_barrier()
        cvec = cntv[pl.ds(0, 16)]
        n = cvec[0]
        nch = cvec[1]

        def gbody(g, carry):
            d1 = pltpu.async_copy(
                srcv.at[pl.ds(g * 128, 128)], gsrc_s.at[offsv.at[g]], sem)
            d2 = pltpu.async_copy(
                dstv.at[pl.ds(g * 128, 128)], gdst_s.at[offsv.at[g]], sem2)
            d1.wait()
            d2.wait()
            return carry
        lax.fori_loop(0, n_grp, gbody, 0)

        pltpu.sync_copy(gsrc_s.at[pl.ds(s * stage, stage)], gsrc)
        pltpu.sync_copy(gdst_s.at[pl.ds(s * stage, stage)], gdst)

        for kk in range(8):
            gsrc[pl.ds(n + kk * 16, 16)] = iot * (v_rows // 16) + kk * 7
            gdst[pl.ds(n + kk * 16, 16)] = jnp.full((16,), TRASH, jnp.int32) + s
        plsc.subcore_barrier()

        def cbody(j, carry):
            pltpu.async_copy(
                y_h.at[gsrc.at[pl.ds(j * 128, 128)]], rows_v, sem).wait()
            for kk in range(8):
                didx = gdst[pl.ds(j * 128 + kk * 16, 16)]
                pltpu.sync_copy(rows_v.at[pl.ds(kk * 16, 16)],
                                acc_s.at[didx], add=True)
            return carry
        lax.fori_loop(0, nch, cbody, 0)
        plsc.subcore_barrier()
        ro = N2 // NS
        pltpu.sync_copy(acc_s.at[pl.ds(s * ro, ro)], stg.at[pl.ds(0, ro)])
        pltpu.sync_copy(stg.at[pl.ds(0, ro)], accp_h.at[c, pl.ds(s * ro, ro)])

    return k(src_t, dst_t, y, offs_t, cnt_t,
             jnp.zeros((ra, D), jnp.float32))


def _tc_b(a1p, y1k, d1k, b1, W2, d2p):
    def body(a_ref, y_ref, d1_ref, b_ref, w_ref, d2_ref, o_ref):
        dis1 = lax.rsqrt(d1_ref[0, :] + d1_ref[1, :] + 2.0)
        acc = a_ref[0] + a_ref[1] + 2.0 * y_ref[...]
        h = jnp.maximum(dis1[:, None] * acc + b_ref[...], 0.0)
        dis2 = lax.rsqrt(d2_ref[0, :] + d2_ref[1, :] + 1.0)
        o_ref[...] = dis2[:, None] * jnp.dot(
            h, w_ref[...], preferred_element_type=jnp.float32)
    return pl.pallas_call(
        body,
        out_shape=jax.ShapeDtypeStruct((N2, D), jnp.float32),
    )(a1p, y1k, d1k, b1, W2, d2p)


def _tc_c(a2p, y2, d2p, b2):
    def body(a_ref, y_ref, d_ref, b_ref, o_ref):
        dis2 = lax.rsqrt(d_ref[0, :] + d_ref[1, :] + 1.0)
        o_ref[...] = dis2[:, None] * (a_ref[0] + a_ref[1] + y_ref[...]) \
            + b_ref[...]
    return pl.pallas_call(
        body,
        out_shape=jax.ShapeDtypeStruct((N2, D), jnp.float32),
    )(a2p, y2, d2p, b2)


def kernel(x, edge_index1, edge_index2, W1, b1, W2, b2):
    ei1 = edge_index1.astype(jnp.int32)
    ei2 = edge_index2.astype(jnp.int32)
    src1, dst1 = ei1[0], ei1[1]
    src2, dst2 = ei2[0], ei2[1]
    x8k = x[:N1]

    ones_t = jnp.ones((E1W // 128, 128), jnp.float32)

    deg1p, deg2p = _deg_call(
        dst1.reshape(NW, E1W // 128, 128),
        dst2.reshape(NW, E2W // 128, 128), ones_t)
    y1, offs1, cnt1, offs2, cnt2 = _tca_offs_call(
        x8k, W1, deg1p,
        src1.reshape(NW, E1W // 128, 128), dst1.reshape(NW, E1W // 128, 128),
        src2.reshape(NW, E2W // 128, 128), dst2.reshape(NW, E2W // 128, 128))
    a1p = _msgpass_call(src1.reshape(NW, E1W), dst1.reshape(NW, E1W),
                        y1, offs1, cnt1.reshape(NW, 1024)[:, :128], E1W, N1)
    y2 = _tc_b(a1p, y1[:N2], deg1p[:, :N2], b1.reshape(1, D), W2, deg2p)
    a2p = _msgpass_call(src2.reshape(NW, E2W), dst2.reshape(NW, E2W),
                        y2, offs2, cnt2.reshape(NW, 1024)[:, :128], E2W, N2)
    return _tc_c(a2p, y2, deg2p, b2.reshape(1, D))
